# baseline (speedup 1.0000x reference)
import functools
import jax
import jax.numpy as jnp
from jax import lax
from jax.experimental import pallas as pl
from jax.experimental.pallas import tpu as pltpu
from jax.experimental.pallas import tpu_sc as plsc

N_NODES = 50000
D = 128
E_EDGES = 300000
E_PAD = 311296
_SCALE = 128.0 ** -0.5

_ROW_BLK = 2000

_NW = 32
_EW = E_PAD // _NW
_CHUNK = 128
_NCH = _EW // _CHUNK
_ET = E_PAD // 16
_SCH = _ET // 4
_NBLK = 8
_BN = 6256
_NPAD = _NBLK * _BN
_CAP = _SCH + 256
_NS = 50176


def _proj_body(x_ref, wq_ref, wk_ref, arel_ref, wv_ref, mrel_ref, prel_ref,
               q_ref, kr_ref, vr_ref):
    x = x_ref[...]
    q_ref[...] = jnp.dot(x, wq_ref[...], preferred_element_type=jnp.float32)
    kr = jnp.dot(jnp.dot(x, wk_ref[...], preferred_element_type=jnp.float32),
                 arel_ref[...], preferred_element_type=jnp.float32)
    kr_ref[...] = kr * (prel_ref[0] * _SCALE)
    vr_ref[...] = jnp.dot(jnp.dot(x, wv_ref[...],
                                  preferred_element_type=jnp.float32),
                          mrel_ref[...], preferred_element_type=jnp.float32)


def _proj(x, wq, wk, arel, wv, mrel, prel):
    n = x.shape[0]
    grid = (n // _ROW_BLK,)
    return pl.pallas_call(
        _proj_body,
        grid=grid,
        in_specs=[
            pl.BlockSpec((_ROW_BLK, D), lambda i: (i, 0)),
            pl.BlockSpec((D, D), lambda i: (0, 0)),
            pl.BlockSpec((D, D), lambda i: (0, 0)),
            pl.BlockSpec((D, D), lambda i: (0, 0)),
            pl.BlockSpec((D, D), lambda i: (0, 0)),
            pl.BlockSpec((D, D), lambda i: (0, 0)),
            pl.BlockSpec(memory_space=pltpu.SMEM),
        ],
        out_specs=[
            pl.BlockSpec((_ROW_BLK, D), lambda i: (i, 0)),
            pl.BlockSpec((_ROW_BLK, D), lambda i: (i, 0)),
            pl.BlockSpec((_ROW_BLK, D), lambda i: (i, 0)),
        ],
        out_shape=[
            jax.ShapeDtypeStruct((n, D), jnp.float32),
            jax.ShapeDtypeStruct((n, D), jnp.float32),
            jax.ShapeDtypeStruct((n, D), jnp.float32),
        ],
    )(x, wq, wk, arel, wv, mrel, jnp.reshape(prel, (1,)))


def _epi_body(num_ref, s0_ref, s1_ref, x_ref, wo_ref, skip_ref,
              out_ref, colsum_ref):
    i = pl.program_id(0)
    s = s0_ref[...] + s1_ref[...]
    o = num_ref[...] / (s + 1e-16)
    o = jax.nn.gelu(o)
    o = jnp.dot(o, wo_ref[...], preferred_element_type=jnp.float32)
    beta = jax.nn.sigmoid(skip_ref[0])
    res = beta * o + (1.0 - beta) * x_ref[...]
    out_ref[...] = res

    @pl.when(i == 0)
    def _():
        colsum_ref[...] = jnp.zeros_like(colsum_ref)

    colsum_ref[...] += jnp.sum(res, axis=0, keepdims=True)


def _epilogue(num, s0, s1, x, wo, skip):
    n = x.shape[0]
    grid = (n // _ROW_BLK,)
    return pl.pallas_call(
        _epi_body,
        grid=grid,
        in_specs=[
            pl.BlockSpec((_ROW_BLK, D), lambda i: (i, 0)),
            pl.BlockSpec((_ROW_BLK, 1), lambda i: (i, 0)),
            pl.BlockSpec((_ROW_BLK, 1), lambda i: (i, 0)),
            pl.BlockSpec((_ROW_BLK, D), lambda i: (i, 0)),
            pl.BlockSpec((D, D), lambda i: (0, 0)),
            pl.BlockSpec(memory_space=pltpu.SMEM),
        ],
        out_specs=[
            pl.BlockSpec((_ROW_BLK, D), lambda i: (i, 0)),
            pl.BlockSpec((1, D), lambda i: (0, 0)),
        ],
        out_shape=[
            jax.ShapeDtypeStruct((n, D), jnp.float32),
            jax.ShapeDtypeStruct((1, D), jnp.float32),
        ],
    )(num, s0, s1, x, wo, jnp.reshape(skip, (1,)))


def _final_body(ca_ref, cb_ref, wl_ref, bl_ref, out_ref):
    pooled = ca_ref[...] + cb_ref[...]
    out_ref[...] = jnp.dot(pooled, wl_ref[...],
                           preferred_element_type=jnp.float32) + bl_ref[...]


def _final(colsum_a, colsum_b, w_lin, b_lin):
    return pl.pallas_call(
        _final_body,
        out_shape=jax.ShapeDtypeStruct((1, 1), jnp.float32),
    )(colsum_a, colsum_b, w_lin, jnp.reshape(b_lin, (1, 1)))


_MESH = None


def _mesh():
    global _MESH
    if _MESH is None:
        _MESH = plsc.VectorSubcoreMesh(core_axis_name="c", subcore_axis_name="s")
    return _MESH


def _p1_compute(base, kr_v, q_v, e_v):
    lane = lax.iota(jnp.int32, 16)

    def edot(g, carry2):
        evec = jnp.zeros((16,), jnp.float32)
        for el2 in range(16):
            el = g * 16 + el2
            acc = kr_v[el, pl.ds(0, 16)] * q_v[el, pl.ds(0, 16)]
            for j in range(1, 8):
                acc = acc + (kr_v[el, pl.ds(j * 16, 16)] *
                             q_v[el, pl.ds(j * 16, 16)])
            evec = jnp.where(lane == el2, jnp.sum(acc), evec)
        ids = base + g * 16 + lane
        e_v[pl.ds(g * 16, 16)] = jnp.where(ids < E_EDGES, jnp.exp(evec), 0.0)
        return carry2

    lax.fori_loop(0, _CHUNK // 16, edot, 0)


def _p1_body(kr_hbm, q_hbm, src_hbm, dst_hbm, z1_hbm, e_hbm, s_hbm,
             src0, dst0, kr0, q0, e0, src1, dst1, kr1, q1, e1, s_sh,
             semk0, semq0, semk1, semq1):
    c = lax.axis_index("c")
    sid = lax.axis_index("s")

    @pl.when(sid == 0)
    def _():
        pltpu.sync_copy(z1_hbm, s_sh)

    plsc.subcore_barrier()

    wid = sid * 2 + c

    def pair(p, carry):
        b0 = wid * _EW + (2 * p) * _CHUNK
        b1 = b0 + _CHUNK
        pltpu.sync_copy(src_hbm.at[pl.ds(b0, _CHUNK)], src0)
        pltpu.sync_copy(dst_hbm.at[pl.ds(b0, _CHUNK)], dst0)
        hk0 = pltpu.async_copy(kr_hbm.at[src0], kr0, semk0)
        hq0 = pltpu.async_copy(q_hbm.at[dst0], q0, semq0)
        pltpu.sync_copy(src_hbm.at[pl.ds(b1, _CHUNK)], src1)
        pltpu.sync_copy(dst_hbm.at[pl.ds(b1, _CHUNK)], dst1)
        hk1 = pltpu.async_copy(kr_hbm.at[src1], kr1, semk1)
        hq1 = pltpu.async_copy(q_hbm.at[dst1], q1, semq1)
        hk0.wait()
        hq0.wait()
        _p1_compute(b0, kr0, q0, e0)
        pltpu.sync_copy(e0, s_sh.at[dst0], add=True)
        pltpu.sync_copy(e0, e_hbm.at[pl.ds(b0, _CHUNK)])
        hk1.wait()
        hq1.wait()
        _p1_compute(b1, kr1, q1, e1)
        pltpu.sync_copy(e1, s_sh.at[dst1], add=True)
        pltpu.sync_copy(e1, e_hbm.at[pl.ds(b1, _CHUNK)])
        return carry

    lax.fori_loop(0, _NCH // 2, pair, 0)
    plsc.subcore_barrier()

    @pl.when(sid == 0)
    def _():
        pltpu.sync_copy(s_sh, s_hbm.at[pl.ds(c * _NS, _NS)])


def _sc_pass1(kr, q, src, dst, z1):
    f = functools.partial(
        pl.kernel,
        mesh=_mesh(),
        compiler_params=pltpu.CompilerParams(needs_layout_passes=False),
        out_type=[
            jax.ShapeDtypeStruct((E_PAD,), jnp.float32),
            jax.ShapeDtypeStruct((2 * _NS,), jnp.float32),
        ],
        scratch_types=(
            [pltpu.VMEM((_CHUNK,), jnp.int32),
             pltpu.VMEM((_CHUNK,), jnp.int32),
             pltpu.VMEM((_CHUNK, D), jnp.float32),
             pltpu.VMEM((_CHUNK, D), jnp.float32),
             pltpu.VMEM((_CHUNK,), jnp.float32)] * 2 +
            [pltpu.VMEM_SHARED((_NS,), jnp.float32),
             pltpu.SemaphoreType.DMA,
             pltpu.SemaphoreType.DMA,
             pltpu.SemaphoreType.DMA,
             pltpu.SemaphoreType.DMA]
        ),
    )(_p1_body)
    return f(kr, q, src, dst, z1)


def _p2_body(vr_hbm, src_hbm, dst_hbm, e_hbm, z2_hbm, num_hbm,
             src_v, dst_v, e_v, cs_v, cd_v, ce_v, didx_v, rows_v, num_sh, sem):
    c = lax.axis_index("c")
    sid = lax.axis_index("s")

    for rnd in range(_NBLK // 2):
        nbase = (rnd * 2 + c) * _BN

        @pl.when(sid == 0)
        def _():
            pltpu.sync_copy(z2_hbm, num_sh)

        plsc.subcore_barrier()

        ebase = sid * _ET

        def schunk(i, carry):
            b = ebase + i * _SCH
            pltpu.sync_copy(src_hbm.at[pl.ds(b, _SCH)], src_v)
            pltpu.sync_copy(dst_hbm.at[pl.ds(b, _SCH)], dst_v)
            pltpu.sync_copy(e_hbm.at[pl.ds(b, _SCH)], e_v)

            def grp(g, cnt2):
                s_g = src_v[pl.ds(g * 16, 16)]
                d_g = dst_v[pl.ds(g * 16, 16)]
                e_g = e_v[pl.ds(g * 16, 16)]
                m = (d_g >= nbase) & (d_g < nbase + _BN)
                plsc.store_compressed(cs_v.at[pl.ds(cnt2, 16)], s_g, mask=m)
                plsc.store_compressed(cd_v.at[pl.ds(cnt2, 16)], d_g - nbase, mask=m)
                plsc.store_compressed(ce_v.at[pl.ds(cnt2, 16)], e_g, mask=m)
                return cnt2 + plsc.all_reduce_population_count(m)[0]

            cnt = lax.fori_loop(0, _SCH // 16, grp, 0)

            zi = jnp.zeros((16,), jnp.int32)
            zf = jnp.zeros((16,), jnp.float32)
            for t in range(_CHUNK // 16):
                cs_v[pl.ds(cnt + t * 16, 16)] = zi
                cd_v[pl.ds(cnt + t * 16, 16)] = zi
                ce_v[pl.ds(cnt + t * 16, 16)] = zf

            nb = (cnt + _CHUNK - 1) // _CHUNK

            def batch(bi, carry2):
                off = bi * _CHUNK
                for t in range(_CHUNK // 16):
                    didx_v[pl.ds(t * 16, 16)] = cd_v[pl.ds(off + t * 16, 16)]
                pltpu.async_copy(vr_hbm.at[cs_v.at[pl.ds(off, _CHUNK)]],
                                 rows_v, sem).wait()

                def scl(el4, carry3):
                    for u in range(4):
                        el = el4 * 4 + u
                        sval = ce_v[pl.ds(off + el, 16)][0]
                        for j in range(8):
                            rows_v[el, pl.ds(j * 16, 16)] = (
                                rows_v[el, pl.ds(j * 16, 16)] * sval)
                    return carry3

                lax.fori_loop(0, _CHUNK // 4, scl, 0)
                pltpu.sync_copy(rows_v, num_sh.at[didx_v], add=True)
                return carry2

            lax.fori_loop(0, nb, batch, 0)
            return carry

        lax.fori_loop(0, _ET // _SCH, schunk, 0)
        plsc.subcore_barrier()

        @pl.when(sid == 0)
        def _():
            pltpu.sync_copy(num_sh, num_hbm.at[pl.ds(nbase, _BN)])

        plsc.subcore_barrier()


def _sc_pass2(vr, src, dst, e, z2):
    f = functools.partial(
        pl.kernel,
        mesh=_mesh(),
        compiler_params=pltpu.CompilerParams(needs_layout_passes=False),
        out_type=jax.ShapeDtypeStruct((_NPAD, D), jnp.float32),
        scratch_types=[
            pltpu.VMEM((_SCH,), jnp.int32),
            pltpu.VMEM((_SCH,), jnp.int32),
            pltpu.VMEM((_SCH,), jnp.float32),
            pltpu.VMEM((_CAP,), jnp.int32),
            pltpu.VMEM((_CAP,), jnp.int32),
            pltpu.VMEM((_CAP,), jnp.float32),
            pltpu.VMEM((_CHUNK,), jnp.int32),
            pltpu.VMEM((_CHUNK, D), jnp.float32),
            pltpu.VMEM_SHARED((_BN, D), jnp.float32),
            pltpu.SemaphoreType.DMA,
        ],
    )(_p2_body)
    return f(vr, src, dst, e, z2)


def kernel(x_a, x_b, params, edge_index_ab, edge_index_ba):
    pad = jnp.zeros((E_PAD - E_EDGES,), jnp.int32)
    edges = {
        'ab': (jnp.concatenate([edge_index_ab[0], pad]),
               jnp.concatenate([edge_index_ab[1], pad])),
        'ba': (jnp.concatenate([edge_index_ba[0], pad]),
               jnp.concatenate([edge_index_ba[1], pad])),
    }
    z1 = jnp.zeros((_NS,), jnp.float32)
    z2 = jnp.zeros((_BN, D), jnp.float32)

    x = {'a': x_a, 'b': x_b}
    colsum = {}
    rel_of = {'a': 'ab', 'b': 'ba'}
    for lp in params['layers']:
        proj = {}
        for t in ('a', 'b'):
            r = rel_of[t]
            proj[t] = _proj(x[t], lp['Wq_' + t], lp['Wk_' + t],
                            lp['Arel_' + r], lp['Wv_' + t], lp['Mrel_' + r],
                            lp['prel_' + r])
        num, s2 = {}, {}
        for (r, st, dt) in (('ab', 'a', 'b'), ('ba', 'b', 'a')):
            src, dst = edges[r]
            q_d = proj[dt][0]
            kr_s = proj[st][1]
            vr_s = proj[st][2]
            e, s2[dt] = _sc_pass1(kr_s, q_d, src, dst, z1)
            num[dt] = lax.slice(_sc_pass2(vr_s, src, dst, e, z2),
                                (0, 0), (N_NODES, D))
        for t in ('a', 'b'):
            s0 = jnp.reshape(s2[t][:N_NODES], (N_NODES, 1))
            s1 = jnp.reshape(s2[t][_NS:_NS + N_NODES], (N_NODES, 1))
            x[t], colsum[t] = _epilogue(num[t], s0, s1, x[t], lp['Wo_' + t],
                                        lp['skip_' + t])
    return _final(colsum['a'], colsum['b'], params['W_lin'], params['b_lin'])


# speedup vs baseline: 1.4507x; 1.4507x over previous
import functools
import jax
import jax.numpy as jnp
from jax import lax
from jax.experimental import pallas as pl
from jax.experimental.pallas import tpu as pltpu
from jax.experimental.pallas import tpu_sc as plsc

N_NODES = 50000
D = 128
E_EDGES = 300000
E_PAD = 311296
_SCALE = 128.0 ** -0.5

_ROW_BLK = 2000

_NW = 32
_EW = E_PAD // _NW
_CHUNK = 128
_NCH = _EW // _CHUNK
_ET = E_PAD // 16
_SCH = _ET // 4
_NBLK = 8
_BN = 6256
_NPAD = _NBLK * _BN
_CAP = _SCH + 128
_B = 96
_NS = 50176


def _proj_body(x_ref, wq_ref, wk_ref, arel_ref, wv_ref, mrel_ref, prel_ref,
               q_ref, kv_ref):
    x = x_ref[...]
    q_ref[...] = jnp.dot(x, wq_ref[...], preferred_element_type=jnp.float32,
                 precision=lax.Precision.HIGHEST)
    kr = jnp.dot(jnp.dot(x, wk_ref[...], preferred_element_type=jnp.float32,
                 precision=lax.Precision.HIGHEST),
                 arel_ref[...], preferred_element_type=jnp.float32,
                 precision=lax.Precision.HIGHEST)
    vr = jnp.dot(jnp.dot(x, wv_ref[...], preferred_element_type=jnp.float32,
                 precision=lax.Precision.HIGHEST),
                 mrel_ref[...], preferred_element_type=jnp.float32,
                 precision=lax.Precision.HIGHEST)
    kv_ref[...] = jnp.concatenate([kr * (prel_ref[0] * _SCALE), vr], axis=1)


def _proj(x, wq, wk, arel, wv, mrel, prel):
    n = x.shape[0]
    grid = (n // _ROW_BLK,)
    return pl.pallas_call(
        _proj_body,
        grid=grid,
        in_specs=[
            pl.BlockSpec((_ROW_BLK, D), lambda i: (i, 0)),
            pl.BlockSpec((D, D), lambda i: (0, 0)),
            pl.BlockSpec((D, D), lambda i: (0, 0)),
            pl.BlockSpec((D, D), lambda i: (0, 0)),
            pl.BlockSpec((D, D), lambda i: (0, 0)),
            pl.BlockSpec((D, D), lambda i: (0, 0)),
            pl.BlockSpec(memory_space=pltpu.SMEM),
        ],
        out_specs=[
            pl.BlockSpec((_ROW_BLK, D), lambda i: (i, 0)),
            pl.BlockSpec((_ROW_BLK, 2 * D), lambda i: (i, 0)),
        ],
        out_shape=[
            jax.ShapeDtypeStruct((n, D), jnp.float32),
            jax.ShapeDtypeStruct((n, 2 * D), jnp.float32),
        ],
    )(x, wq, wk, arel, wv, mrel, jnp.reshape(prel, (1,)))


def _epi_body(num_ref, s0_ref, s1_ref, x_ref, wo_ref, skip_ref,
              out_ref, colsum_ref):
    i = pl.program_id(0)
    s = s0_ref[...] + s1_ref[...]
    o = num_ref[...] / (s + 1e-16)
    o = jax.nn.gelu(o)
    o = jnp.dot(o, wo_ref[...], preferred_element_type=jnp.float32,
                 precision=lax.Precision.HIGHEST)
    beta = jax.nn.sigmoid(skip_ref[0])
    res = beta * o + (1.0 - beta) * x_ref[...]
    out_ref[...] = res

    @pl.when(i == 0)
    def _():
        colsum_ref[...] = jnp.zeros_like(colsum_ref)

    colsum_ref[...] += jnp.sum(res, axis=0, keepdims=True)


def _epilogue(num, s0, s1, x, wo, skip):
    n = x.shape[0]
    grid = (n // _ROW_BLK,)
    return pl.pallas_call(
        _epi_body,
        grid=grid,
        in_specs=[
            pl.BlockSpec((_ROW_BLK, D), lambda i: (i, 0)),
            pl.BlockSpec((_ROW_BLK, 1), lambda i: (i, 0)),
            pl.BlockSpec((_ROW_BLK, 1), lambda i: (i, 0)),
            pl.BlockSpec((_ROW_BLK, D), lambda i: (i, 0)),
            pl.BlockSpec((D, D), lambda i: (0, 0)),
            pl.BlockSpec(memory_space=pltpu.SMEM),
        ],
        out_specs=[
            pl.BlockSpec((_ROW_BLK, D), lambda i: (i, 0)),
            pl.BlockSpec((1, D), lambda i: (0, 0)),
        ],
        out_shape=[
            jax.ShapeDtypeStruct((n, D), jnp.float32),
            jax.ShapeDtypeStruct((1, D), jnp.float32),
        ],
    )(num, s0, s1, x, wo, jnp.reshape(skip, (1,)))


def _final_body(ca_ref, cb_ref, wl_ref, bl_ref, out_ref):
    pooled = ca_ref[...] + cb_ref[...]
    out_ref[...] = jnp.dot(pooled, wl_ref[...],
                           preferred_element_type=jnp.float32,
                 precision=lax.Precision.HIGHEST) + bl_ref[...]


def _final(colsum_a, colsum_b, w_lin, b_lin):
    return pl.pallas_call(
        _final_body,
        out_shape=jax.ShapeDtypeStruct((1, 1), jnp.float32),
    )(colsum_a, colsum_b, w_lin, jnp.reshape(b_lin, (1, 1)))


_MESH = None


def _mesh():
    global _MESH
    if _MESH is None:
        _MESH = plsc.VectorSubcoreMesh(core_axis_name="c", subcore_axis_name="s")
    return _MESH


def _pf_body(kv_hbm, q_hbm, src_hbm, dst_hbm, z2_hbm, z1_hbm,
             num_hbm, s_hbm,
             ssrc, sdst, csrc, cdoff, didx, didx2, ev_b,
             kv_rows, q_rows, rows_v, num_sh, s_sh, semkv, semq):
    c = lax.axis_index("c")
    sid = lax.axis_index("s")
    lane = lax.iota(jnp.int32, 16)

    @pl.when(sid == 0)
    def _():
        pltpu.sync_copy(z1_hbm, s_sh)

    for rnd in range(_NBLK // 2):
        nbase = (rnd * 2 + c) * _BN

        @pl.when(sid == 0)
        def _():
            pltpu.sync_copy(z2_hbm, num_sh)

        plsc.subcore_barrier()

        ebase = sid * _ET

        def schunk(i, carry):
            b = ebase + i * _SCH
            pltpu.sync_copy(src_hbm.at[pl.ds(b, _SCH)], ssrc)
            pltpu.sync_copy(dst_hbm.at[pl.ds(b, _SCH)], sdst)

            def grp(g, cnt2):
                s_g = ssrc[pl.ds(g * 16, 16)]
                d_g = sdst[pl.ds(g * 16, 16)]
                m = (d_g >= nbase) & (d_g < nbase + _BN)
                plsc.store_compressed(csrc.at[pl.ds(cnt2, 16)], s_g, mask=m)
                plsc.store_compressed(cdoff.at[pl.ds(cnt2, 16)], d_g - nbase,
                                      mask=m)
                return cnt2 + plsc.all_reduce_population_count(m)[0]

            cnt = lax.fori_loop(0, _SCH // 16, grp, 0)

            zi = jnp.zeros((16,), jnp.int32)
            for t in range(_B // 16):
                csrc[pl.ds(cnt + t * 16, 16)] = zi
                cdoff[pl.ds(cnt + t * 16, 16)] = zi

            nb = (cnt + _B - 1) // _B

            def batch(bi, carry2):
                off = bi * _B
                for t in range(_B // 16):
                    dd = cdoff[pl.ds(off + t * 16, 16)]
                    didx[pl.ds(t * 16, 16)] = dd
                    didx2[pl.ds(t * 16, 16)] = dd + nbase
                hkv = pltpu.async_copy(kv_hbm.at[csrc.at[pl.ds(off, _B)]],
                                       kv_rows, semkv)
                hq = pltpu.async_copy(q_hbm.at[didx2], q_rows, semq)
                hkv.wait()
                hq.wait()

                def egrp(g, carry3):
                    evec = jnp.zeros((16,), jnp.float32)
                    for el2 in range(16):
                        el = g * 16 + el2
                        acc = (kv_rows[el, pl.ds(0, 16)] *
                               q_rows[el, pl.ds(0, 16)])
                        for j in range(1, 8):
                            acc = acc + (kv_rows[el, pl.ds(j * 16, 16)] *
                                         q_rows[el, pl.ds(j * 16, 16)])
                        evec = jnp.where(lane == el2, jnp.sum(acc), evec)
                    pos = off + g * 16 + lane
                    ev_b[pl.ds(g * 16, 16)] = jnp.where(pos < cnt,
                                                        jnp.exp(evec), 0.0)
                    for el2 in range(16):
                        el = g * 16 + el2
                        sval = ev_b[pl.ds(g * 16 + el2, 16)][0]
                        for j in range(8):
                            rows_v[el, pl.ds(j * 16, 16)] = (
                                kv_rows[el, pl.ds(D + j * 16, 16)] * sval)
                    return carry3

                lax.fori_loop(0, _B // 16, egrp, 0)
                pltpu.sync_copy(rows_v, num_sh.at[didx], add=True)
                pltpu.sync_copy(ev_b, s_sh.at[didx2], add=True)
                return carry2

            lax.fori_loop(0, nb, batch, 0)
            return carry

        lax.fori_loop(0, _ET // _SCH, schunk, 0)
        plsc.subcore_barrier()

        @pl.when(sid == 0)
        def _():
            pltpu.sync_copy(num_sh, num_hbm.at[pl.ds(nbase, _BN)])

        plsc.subcore_barrier()

    @pl.when(sid == 0)
    def _():
        pltpu.sync_copy(s_sh, s_hbm.at[pl.ds(c * _NS, _NS)])


def _sc_fused(kv, q, src, dst, z2, z1):
    f = functools.partial(
        pl.kernel,
        mesh=_mesh(),
        compiler_params=pltpu.CompilerParams(needs_layout_passes=False),
        out_type=[
            jax.ShapeDtypeStruct((_NPAD, D), jnp.float32),
            jax.ShapeDtypeStruct((2 * _NS,), jnp.float32),
        ],
        scratch_types=[
            pltpu.VMEM((_SCH,), jnp.int32),
            pltpu.VMEM((_SCH,), jnp.int32),
            pltpu.VMEM((_CAP,), jnp.int32),
            pltpu.VMEM((_CAP,), jnp.int32),
            pltpu.VMEM((_B,), jnp.int32),
            pltpu.VMEM((_B,), jnp.int32),
            pltpu.VMEM((_B,), jnp.float32),
            pltpu.VMEM((_B, 2 * D), jnp.float32),
            pltpu.VMEM((_B, D), jnp.float32),
            pltpu.VMEM((_B, D), jnp.float32),
            pltpu.VMEM_SHARED((_BN, D), jnp.float32),
            pltpu.VMEM_SHARED((_NS,), jnp.float32),
            pltpu.SemaphoreType.DMA,
            pltpu.SemaphoreType.DMA,
        ],
    )(_pf_body)
    return f(kv, q, src, dst, z2, z1)


def kernel(x_a, x_b, params, edge_index_ab, edge_index_ba):
    pad = jnp.zeros((E_PAD - E_EDGES,), jnp.int32)
    padd = jnp.full((E_PAD - E_EDGES,), 60000, jnp.int32)
    edges = {
        'ab': (jnp.concatenate([edge_index_ab[0], pad]),
               jnp.concatenate([edge_index_ab[1], padd])),
        'ba': (jnp.concatenate([edge_index_ba[0], pad]),
               jnp.concatenate([edge_index_ba[1], padd])),
    }
    z1 = jnp.zeros((_NS,), jnp.float32)
    z2 = jnp.zeros((_BN, D), jnp.float32)

    x = {'a': x_a, 'b': x_b}
    colsum = {}
    rel_of = {'a': 'ab', 'b': 'ba'}
    for lp in params['layers']:
        proj = {}
        for t in ('a', 'b'):
            r = rel_of[t]
            proj[t] = _proj(x[t], lp['Wq_' + t], lp['Wk_' + t],
                            lp['Arel_' + r], lp['Wv_' + t], lp['Mrel_' + r],
                            lp['prel_' + r])
        num, s2 = {}, {}
        for (r, st, dt) in (('ab', 'a', 'b'), ('ba', 'b', 'a')):
            esrc, edst = edges[r]
            q_d = proj[dt][0]
            kv_s = proj[st][1]
            nfull, s2[dt] = _sc_fused(kv_s, q_d, esrc, edst, z2, z1)
            num[dt] = lax.slice(nfull, (0, 0), (N_NODES, D))
        for t in ('a', 'b'):
            s0 = jnp.reshape(s2[t][:N_NODES], (N_NODES, 1))
            s1 = jnp.reshape(s2[t][_NS:_NS + N_NODES], (N_NODES, 1))
            x[t], colsum[t] = _epilogue(num[t], s0, s1, x[t], lp['Wo_' + t],
                                        lp['skip_' + t])
    return _final(colsum['a'], colsum['b'], params['W_lin'], params['b_lin'])
